# baseline (speedup 1.0000x reference)
import functools

import jax
import jax.numpy as jnp
from jax import lax
from jax.experimental import pallas as pl
from jax.experimental.pallas import tpu as pltpu
from jax.experimental.pallas import tpu_sc as plsc

_NUM_BLOCKS = 4
_ASPECT = 0.75
_NC = 2
_NS = 16
_L = 16


def _make_sc_call(batch, height, width):
    seq = height * width
    nw = _NC * _NS
    bpw = batch // nw
    n_chunks = width // _L

    mesh = plsc.VectorSubcoreMesh(core_axis_name="c", subcore_axis_name="s")

    @functools.partial(
        pl.kernel,
        mesh=mesh,
        out_type=[
            jax.ShapeDtypeStruct((batch, seq), jnp.int32),
            jax.ShapeDtypeStruct((batch, seq), jnp.int32),
        ],
        scratch_types=[
            pltpu.VMEM((4 * _NUM_BLOCKS, _L), jnp.int32),
            pltpu.VMEM((seq,), jnp.int32),
            pltpu.VMEM((seq,), jnp.int32),
            pltpu.VMEM((seq + _L,), jnp.int32),
            pltpu.VMEM((seq + _L,), jnp.int32),
            pltpu.SemaphoreType.DMA,
            pltpu.SemaphoreType.DMA,
        ],
        compiler_params=pltpu.CompilerParams(needs_layout_passes=False),
    )
    def sc_call(params_hbm, mask_hbm, pos_hbm, wparams_v, mask_v0, mask_v1,
                pos_v0, pos_v1, sem0, sem1):
        wid = lax.axis_index("s") * _NC + lax.axis_index("c")
        lane = lax.iota(jnp.int32, _L)
        neg1 = jnp.full((_L,), -1, jnp.int32)
        sems = [sem0, sem1]
        mask_bufs = [mask_v0, mask_v1]
        pos_bufs = [pos_v0, pos_v1]
        pending = [None, None]

        for bi in range(bpw):
            buf = bi % 2
            mask_v = mask_bufs[buf]
            pos_v = pos_bufs[buf]
            b = wid * bpw + bi
            pltpu.sync_copy(params_hbm.at[b], wparams_v)

            def bcast(col):
                return wparams_v[col, :]

            tops = [bcast(k) for k in range(_NUM_BLOCKS)]
            bm1 = [bcast(_NUM_BLOCKS + k) for k in range(_NUM_BLOCKS)]
            colsgn = []
            for k in range(_NUM_BLOCKS):
                lk = bcast(2 * _NUM_BLOCKS + k)
                rk1 = bcast(3 * _NUM_BLOCKS + k)
                colsgn.append([
                    (((lane + j * _L) - lk) | (rk1 - (lane + j * _L))) >> 31
                    for j in range(n_chunks)
                ])
            if pending[buf] is not None:
                for h in pending[buf]:
                    h.wait()
                pending[buf] = None

            def row_body(r, cnt):
                rowsgn = [
                    ((r - tops[k]) | (bm1[k] - r)) >> 31
                    for k in range(_NUM_BLOCKS)
                ]
                for j in range(n_chunks):
                    base = r * width + j * _L
                    s = rowsgn[0] | colsgn[0][j]
                    for k in range(1, _NUM_BLOCKS):
                        s = s & (rowsgn[k] | colsgn[k][j])
                    mi = s + 1
                    mask_v[pl.ds(base, _L)] = mi
                    plsc.store_compressed(
                        pos_v.at[pl.ds(cnt, _L)], lane + base, mask=mi > 0)
                    pc = plsc.all_reduce_population_count(mi > 0)
                    cnt = cnt + pc[0]
                return cnt

            cnt = lax.fori_loop(0, height, row_body, jnp.int32(0))
            def fill_body(t, c):
                pos_v[pl.ds(c, _L)] = neg1
                return c + _L

            lax.fori_loop(0, (seq - cnt + _L - 1) // _L, fill_body, cnt)
            h0 = pltpu.async_copy(mask_v, mask_hbm.at[b], sems[buf])
            h1 = pltpu.async_copy(pos_v.at[pl.ds(0, seq)], pos_hbm.at[b], sems[buf])
            pending[buf] = (h0, h1)

        for p in pending:
            if p is not None:
                for h in p:
                    h.wait()

    return sc_call


def kernel(batch_size, seq_len, height, width, scales, rand_top, rand_left):
    height_static = 64
    width_static = 64
    batch_static = scales.shape[0] // _NUM_BLOCKS

    areas = (scales * height * width).astype(jnp.int32)
    hs = jnp.clip(
        jnp.sqrt(areas.astype(jnp.float32) / _ASPECT).astype(jnp.int32),
        1, height)
    ws = jnp.clip((areas / jnp.clip(hs, 1, None)).astype(jnp.int32), 1, width)
    max_tops = jnp.clip(height - hs + 1, 1, None)
    max_lefts = jnp.clip(width - ws + 1, 1, None)
    tops = (rand_top * max_tops.astype(jnp.float32)).astype(jnp.int32)
    lefts = (rand_left * max_lefts.astype(jnp.float32)).astype(jnp.int32)

    b = batch_static
    k = _NUM_BLOCKS
    params = jnp.stack(
        [tops.reshape(b, k), (tops + hs - 1).reshape(b, k),
         lefts.reshape(b, k), (lefts + ws - 1).reshape(b, k)], axis=1)
    params = jnp.broadcast_to(
        params.reshape(b, 4 * k, 1), (b, 4 * k, _L)).astype(jnp.int32)

    sc_call = _make_sc_call(batch_static, height_static, width_static)
    mask_i, positions = sc_call(params)

    target_mask = mask_i.astype(bool)
    return (jnp.logical_not(target_mask), target_mask, positions)


# speedup vs baseline: 1.1155x; 1.1155x over previous
import functools

import jax
import jax.numpy as jnp
from jax import lax
from jax.experimental import pallas as pl
from jax.experimental.pallas import tpu as pltpu
from jax.experimental.pallas import tpu_sc as plsc

_NUM_BLOCKS = 4
_ASPECT = 0.75
_NC = 2
_NS = 16
_L = 16


def _make_sc_call(batch, height, width):
    seq = height * width
    nw = _NC * _NS
    bpw = batch // nw
    n_chunks = width // _L

    mesh = plsc.VectorSubcoreMesh(core_axis_name="c", subcore_axis_name="s")

    @functools.partial(
        pl.kernel,
        mesh=mesh,
        out_type=[
            jax.ShapeDtypeStruct((batch, seq), jnp.int32),
            jax.ShapeDtypeStruct((batch, seq), jnp.int32),
        ],
        scratch_types=[
            pltpu.VMEM((4 * _NUM_BLOCKS, _L), jnp.int32),
            pltpu.VMEM((seq,), jnp.int32),
            pltpu.VMEM((seq,), jnp.int32),
            pltpu.VMEM((seq + _L,), jnp.int32),
            pltpu.VMEM((seq + _L,), jnp.int32),
            pltpu.SemaphoreType.DMA,
            pltpu.SemaphoreType.DMA,
        ],
        compiler_params=pltpu.CompilerParams(needs_layout_passes=False),
    )
    def sc_call(params_hbm, mask_hbm, pos_hbm, wparams_v, mask_v0, mask_v1,
                pos_v0, pos_v1, sem0, sem1):
        wid = lax.axis_index("s") * _NC + lax.axis_index("c")
        lane = lax.iota(jnp.int32, _L)
        neg1 = jnp.full((_L,), -1, jnp.int32)
        sems = [sem0, sem1]
        mask_bufs = [mask_v0, mask_v1]
        pos_bufs = [pos_v0, pos_v1]
        pending = [None, None]

        for bi in range(bpw):
            buf = bi % 2
            mask_v = mask_bufs[buf]
            pos_v = pos_bufs[buf]
            b = wid * bpw + bi
            pltpu.sync_copy(params_hbm.at[b], wparams_v)

            def bcast(col):
                return wparams_v[col, :]

            tops = [bcast(k) for k in range(_NUM_BLOCKS)]
            bm1 = [bcast(_NUM_BLOCKS + k) for k in range(_NUM_BLOCKS)]
            colsgn = []
            for k in range(_NUM_BLOCKS):
                lk = bcast(2 * _NUM_BLOCKS + k)
                rk1 = bcast(3 * _NUM_BLOCKS + k)
                colsgn.append([
                    (((lane + j * _L) - lk) | (rk1 - (lane + j * _L))) >> 31
                    for j in range(n_chunks)
                ])
            if pending[buf] is not None:
                for h in pending[buf]:
                    h.wait()
                pending[buf] = None

            def row_body(r, cnt):
                rowsgn = [
                    ((r - tops[k]) | (bm1[k] - r)) >> 31
                    for k in range(_NUM_BLOCKS)
                ]
                for j in range(n_chunks):
                    base = r * width + j * _L
                    s = rowsgn[0] | colsgn[0][j]
                    for k in range(1, _NUM_BLOCKS):
                        s = s & (rowsgn[k] | colsgn[k][j])
                    mi = s + 1
                    mask_v[pl.ds(base, _L)] = mi
                    pos_v[pl.ds(base, _L)] = neg1
                    plsc.store_compressed(
                        pos_v.at[pl.ds(cnt, _L)], lane + base, mask=mi > 0)
                    pc = plsc.all_reduce_population_count(mi > 0)
                    cnt = cnt + pc[0]
                return cnt

            lax.fori_loop(0, height, row_body, jnp.int32(0))
            h0 = pltpu.async_copy(mask_v, mask_hbm.at[b], sems[buf])
            h1 = pltpu.async_copy(pos_v.at[pl.ds(0, seq)], pos_hbm.at[b], sems[buf])
            pending[buf] = (h0, h1)

        for p in pending:
            if p is not None:
                for h in p:
                    h.wait()

    return sc_call


def kernel(batch_size, seq_len, height, width, scales, rand_top, rand_left):
    height_static = 64
    width_static = 64
    batch_static = scales.shape[0] // _NUM_BLOCKS

    areas = (scales * height * width).astype(jnp.int32)
    hs = jnp.clip(
        jnp.sqrt(areas.astype(jnp.float32) / _ASPECT).astype(jnp.int32),
        1, height)
    ws = jnp.clip((areas / jnp.clip(hs, 1, None)).astype(jnp.int32), 1, width)
    max_tops = jnp.clip(height - hs + 1, 1, None)
    max_lefts = jnp.clip(width - ws + 1, 1, None)
    tops = (rand_top * max_tops.astype(jnp.float32)).astype(jnp.int32)
    lefts = (rand_left * max_lefts.astype(jnp.float32)).astype(jnp.int32)

    b = batch_static
    k = _NUM_BLOCKS
    params = jnp.stack(
        [tops.reshape(b, k), (tops + hs - 1).reshape(b, k),
         lefts.reshape(b, k), (lefts + ws - 1).reshape(b, k)], axis=1)
    params = jnp.broadcast_to(
        params.reshape(b, 4 * k, 1), (b, 4 * k, _L)).astype(jnp.int32)

    sc_call = _make_sc_call(batch_static, height_static, width_static)
    mask_i, positions = sc_call(params)

    target_mask = mask_i.astype(bool)
    return (jnp.logical_not(target_mask), target_mask, positions)


# speedup vs baseline: 1.3741x; 1.2319x over previous
import functools

import jax
import jax.numpy as jnp
from jax import lax
from jax.experimental import pallas as pl
from jax.experimental.pallas import tpu as pltpu
from jax.experimental.pallas import tpu_sc as plsc

_NUM_BLOCKS = 4
_ASPECT = 0.75
_NC = 2
_NS = 16
_L = 16


def _make_sc_call(batch, height, width):
    seq = height * width
    nw = _NC * _NS
    bpw = batch // nw
    n_chunks = width // _L

    mesh = plsc.VectorSubcoreMesh(core_axis_name="c", subcore_axis_name="s")

    @functools.partial(
        pl.kernel,
        mesh=mesh,
        out_type=[
            jax.ShapeDtypeStruct((batch, seq), jnp.int32),
            jax.ShapeDtypeStruct((batch, seq), jnp.int32),
        ],
        scratch_types=[
            pltpu.VMEM((batch // (_NC * _NS), 4 * _NUM_BLOCKS), jnp.int32),
            pltpu.VMEM((seq,), jnp.int32),
            pltpu.VMEM((seq,), jnp.int32),
            pltpu.VMEM((seq + _L,), jnp.int32),
            pltpu.VMEM((seq + _L,), jnp.int32),
            pltpu.SemaphoreType.DMA,
            pltpu.SemaphoreType.DMA,
        ],
        compiler_params=pltpu.CompilerParams(needs_layout_passes=False),
    )
    def sc_call(params_hbm, mask_hbm, pos_hbm, wparams_v, mask_v0, mask_v1,
                pos_v0, pos_v1, sem0, sem1):
        wid = lax.axis_index("s") * _NC + lax.axis_index("c")
        lane = lax.iota(jnp.int32, _L)
        neg1 = jnp.full((_L,), -1, jnp.int32)
        sems = [sem0, sem1]
        mask_bufs = [mask_v0, mask_v1]
        pos_bufs = [pos_v0, pos_v1]
        pending = [None, None]

        pltpu.sync_copy(params_hbm.at[pl.ds(wid * bpw, bpw)], wparams_v)

        for bi in range(bpw):
            buf = bi % 2
            mask_v = mask_bufs[buf]
            pos_v = pos_bufs[buf]
            b = wid * bpw + bi

            prow = wparams_v[bi, :]

            def bcast(col):
                return prow[col]

            tops = [bcast(k) for k in range(_NUM_BLOCKS)]
            bm1 = [bcast(_NUM_BLOCKS + k) for k in range(_NUM_BLOCKS)]
            colsgn = []
            for k in range(_NUM_BLOCKS):
                lk = bcast(2 * _NUM_BLOCKS + k)
                rk1 = bcast(3 * _NUM_BLOCKS + k)
                colsgn.append([
                    (((lane + j * _L) - lk) | (rk1 - (lane + j * _L))) >> 31
                    for j in range(n_chunks)
                ])
            if pending[buf] is not None:
                for h in pending[buf]:
                    h.wait()
                pending[buf] = None

            def row_body(r, cnt):
                rowsgn = [
                    ((r - tops[k]) | (bm1[k] - r)) >> 31
                    for k in range(_NUM_BLOCKS)
                ]
                for j in range(n_chunks):
                    base = r * width + j * _L
                    s = rowsgn[0] | colsgn[0][j]
                    for k in range(1, _NUM_BLOCKS):
                        s = s & (rowsgn[k] | colsgn[k][j])
                    mi = s + 1
                    mask_v[pl.ds(base, _L)] = mi
                    pos_v[pl.ds(base, _L)] = neg1
                    plsc.store_compressed(
                        pos_v.at[pl.ds(cnt, _L)], lane + base, mask=mi > 0)
                    pc = plsc.all_reduce_population_count(mi > 0)
                    cnt = cnt + pc[0]
                return cnt

            lax.fori_loop(0, height, row_body, jnp.int32(0))
            h0 = pltpu.async_copy(mask_v, mask_hbm.at[b], sems[buf])
            h1 = pltpu.async_copy(pos_v.at[pl.ds(0, seq)], pos_hbm.at[b], sems[buf])
            pending[buf] = (h0, h1)

        for p in pending:
            if p is not None:
                for h in p:
                    h.wait()

    return sc_call


def kernel(batch_size, seq_len, height, width, scales, rand_top, rand_left):
    height_static = 64
    width_static = 64
    batch_static = scales.shape[0] // _NUM_BLOCKS

    areas = (scales * height * width).astype(jnp.int32)
    hs = jnp.clip(
        jnp.sqrt(areas.astype(jnp.float32) / _ASPECT).astype(jnp.int32),
        1, height)
    ws = jnp.clip((areas / jnp.clip(hs, 1, None)).astype(jnp.int32), 1, width)
    max_tops = jnp.clip(height - hs + 1, 1, None)
    max_lefts = jnp.clip(width - ws + 1, 1, None)
    tops = (rand_top * max_tops.astype(jnp.float32)).astype(jnp.int32)
    lefts = (rand_left * max_lefts.astype(jnp.float32)).astype(jnp.int32)

    b = batch_static
    k = _NUM_BLOCKS
    params = jnp.stack(
        [tops.reshape(b, k), (tops + hs - 1).reshape(b, k),
         lefts.reshape(b, k), (lefts + ws - 1).reshape(b, k)],
        axis=1).reshape(b, 4 * k).astype(jnp.int32)

    sc_call = _make_sc_call(batch_static, height_static, width_static)
    mask_i, positions = sc_call(params)

    target_mask = mask_i.astype(bool)
    return (jnp.logical_not(target_mask), target_mask, positions)


# speedup vs baseline: 1.3786x; 1.0032x over previous
import functools

import jax
import jax.numpy as jnp
from jax import lax
from jax.experimental import pallas as pl
from jax.experimental.pallas import tpu as pltpu
from jax.experimental.pallas import tpu_sc as plsc

_NUM_BLOCKS = 4
_ASPECT = 0.75
_NC = 2
_NS = 16
_L = 16


def _make_sc_call(batch, height, width):
    seq = height * width
    nw = _NC * _NS
    bpw = batch // nw
    n_chunks = width // _L

    mesh = plsc.VectorSubcoreMesh(core_axis_name="c", subcore_axis_name="s")

    @functools.partial(
        pl.kernel,
        mesh=mesh,
        out_type=[
            jax.ShapeDtypeStruct((batch, seq), jnp.int32),
            jax.ShapeDtypeStruct((batch, seq), jnp.int32),
        ],
        scratch_types=(
            [pltpu.VMEM((batch // (_NC * _NS), 4 * _NUM_BLOCKS), jnp.int32)]
            + [pltpu.VMEM((seq,), jnp.int32) for _ in range(batch // (_NC * _NS))]
            + [pltpu.VMEM((seq + _L,), jnp.int32) for _ in range(batch // (_NC * _NS))]
            + [pltpu.SemaphoreType.DMA, pltpu.SemaphoreType.DMA]
        ),
        compiler_params=pltpu.CompilerParams(needs_layout_passes=False),
    )
    def sc_call(params_hbm, mask_hbm, pos_hbm, wparams_v, *bufs):
        mask_bufs = bufs[:bpw]
        pos_bufs = bufs[bpw:2 * bpw]
        msem, psem = bufs[2 * bpw], bufs[2 * bpw + 1]
        wid = lax.axis_index("s") * _NC + lax.axis_index("c")
        lane = lax.iota(jnp.int32, _L)
        neg1 = jnp.full((_L,), -1, jnp.int32)
        handles = []

        pltpu.sync_copy(params_hbm.at[pl.ds(wid * bpw, bpw)], wparams_v)

        for pi in range(bpw // 2):
            b0, b1 = 2 * pi, 2 * pi + 1
            sub = []
            for bi in (b0, b1):
                prow = wparams_v[bi, :]
                par = [prow[c] for c in range(4 * _NUM_BLOCKS)]
                tops = par[0:4]
                bm1 = par[4:8]
                colsgn = [[
                    ((((lane + j * _L) - par[8 + k])
                      | (par[12 + k] - (lane + j * _L))) >> 31)
                    for j in range(n_chunks)] for k in range(_NUM_BLOCKS)]
                sub.append((tops, bm1, colsgn))

            def row_body(r, cnts):
                new = []
                for (tops, bm1, colsgn), cnt, mask_v, pos_v in zip(
                        sub, cnts,
                        (mask_bufs[b0], mask_bufs[b1]),
                        (pos_bufs[b0], pos_bufs[b1])):
                    rowsgn = [
                        ((r - tops[k]) | (bm1[k] - r)) >> 31
                        for k in range(_NUM_BLOCKS)
                    ]
                    for j in range(n_chunks):
                        base = r * width + j * _L
                        s = rowsgn[0] | colsgn[0][j]
                        for k in range(1, _NUM_BLOCKS):
                            s = s & (rowsgn[k] | colsgn[k][j])
                        mi = s + 1
                        mask_v[pl.ds(base, _L)] = mi
                        pos_v[pl.ds(base, _L)] = neg1
                        plsc.store_compressed(
                            pos_v.at[pl.ds(cnt, _L)], lane + base,
                            mask=mi > 0)
                        pc = plsc.all_reduce_population_count(mi > 0)
                        cnt = cnt + pc[0]
                    new.append(cnt)
                return tuple(new)

            lax.fori_loop(0, height, row_body,
                          (jnp.int32(0), jnp.int32(0)))
            for bi in (b0, b1):
                b = wid * bpw + bi
                handles.append(
                    pltpu.async_copy(mask_bufs[bi], mask_hbm.at[b], msem))
                handles.append(
                    pltpu.async_copy(pos_bufs[bi].at[pl.ds(0, seq)],
                                     pos_hbm.at[b], psem))

        for h in handles:
            h.wait()

    return sc_call


def kernel(batch_size, seq_len, height, width, scales, rand_top, rand_left):
    height_static = 64
    width_static = 64
    batch_static = scales.shape[0] // _NUM_BLOCKS

    areas = (scales * height * width).astype(jnp.int32)
    hs = jnp.clip(
        jnp.sqrt(areas.astype(jnp.float32) / _ASPECT).astype(jnp.int32),
        1, height)
    ws = jnp.clip((areas / jnp.clip(hs, 1, None)).astype(jnp.int32), 1, width)
    max_tops = jnp.clip(height - hs + 1, 1, None)
    max_lefts = jnp.clip(width - ws + 1, 1, None)
    tops = (rand_top * max_tops.astype(jnp.float32)).astype(jnp.int32)
    lefts = (rand_left * max_lefts.astype(jnp.float32)).astype(jnp.int32)

    b = batch_static
    k = _NUM_BLOCKS
    params = jnp.stack(
        [tops.reshape(b, k), (tops + hs - 1).reshape(b, k),
         lefts.reshape(b, k), (lefts + ws - 1).reshape(b, k)],
        axis=1).reshape(b, 4 * k).astype(jnp.int32)

    sc_call = _make_sc_call(batch_static, height_static, width_static)
    mask_i, positions = sc_call(params)

    target_mask = mask_i.astype(bool)
    return (jnp.logical_not(target_mask), target_mask, positions)
